# baseline (speedup 1.0000x reference)
import functools
import math

import jax
import jax.numpy as jnp
from jax import lax
from jax.experimental import pallas as pl
from jax.experimental.pallas import tpu as pltpu
from jax.experimental.pallas import tpu_sc as plsc

_NC = 2
_NS = 16
_NW = _NC * _NS
_G = 8
_NBUF = 6


def _build_sc_gather_mean(d, s, rows_pad, rpw):
    gs = _G * s
    ng = rpw // _G
    nchunk = d // 16
    mesh = plsc.VectorSubcoreMesh(core_axis_name="c", subcore_axis_name="s")

    @functools.partial(
        pl.kernel,
        out_type=jax.ShapeDtypeStruct((rows_pad, d), jnp.float32),
        mesh=mesh,
        scratch_types=(
            [pltpu.VMEM((rpw * s,), jnp.int32)]
            + [pltpu.VMEM((gs, d), jnp.float32) for _ in range(_NBUF)]
            + [pltpu.VMEM((_G, d), jnp.float32) for _ in range(_NBUF)]
            + [pltpu.SemaphoreType.DMA for _ in range(2 * _NBUF)]
        ),
    )
    def sc_gather_mean(feat, idx, out, idx_v, *bufs):
        gbs = bufs[:_NBUF]
        obs = bufs[_NBUF:2 * _NBUF]
        gss = bufs[2 * _NBUF:3 * _NBUF]
        oss = bufs[3 * _NBUF:]
        wid = lax.axis_index("s") * _NC + lax.axis_index("c")
        row0 = wid * rpw
        pltpu.sync_copy(idx.at[pl.ds(row0 * s, rpw * s)], idx_v)

        def gather_desc(gi, gb, sem):
            off = pl.multiple_of(gi * gs, gs)
            return pltpu.make_async_copy(feat.at[idx_v.at[pl.ds(off, gs)]], gb, sem)

        def out_desc(gi, ob, sem):
            off = pl.multiple_of(row0 + gi * _G, _G)
            return pltpu.make_async_copy(ob, out.at[pl.ds(off, _G)], sem)

        def accumulate(gb, ob):
            def per_dest(di, carry):
                accs = [gb[di * s, pl.ds(ci * 16, 16)] for ci in range(nchunk)]
                for si in range(1, s):
                    for ci in range(nchunk):
                        accs[ci] = accs[ci] + gb[di * s + si, pl.ds(ci * 16, 16)]
                for ci in range(nchunk):
                    ob[di, pl.ds(ci * 16, 16)] = accs[ci]
                return carry

            lax.fori_loop(0, _G, per_dest, 0)

        def process(gi, k, first):
            gather_desc(gi, gbs[k], gss[k]).wait()
            if not first:
                out_desc(gi, obs[k], oss[k]).wait()
            accumulate(gbs[k], obs[k])
            nxt = gi + _NBUF if first else jnp.minimum(gi + _NBUF, ng - 1)
            gather_desc(nxt, gbs[k], gss[k]).start()
            out_desc(gi, obs[k], oss[k]).start()

        for k in range(_NBUF):
            gather_desc(k, gbs[k], gss[k]).start()
        for k in range(_NBUF):
            process(k, k, True)

        def body(t, carry):
            gi = t * _NBUF
            for k in range(_NBUF):
                process(gi + k, k, False)
            return carry

        lax.fori_loop(1, ng // _NBUF, body, 0)
        for k in range(_NBUF):
            gather_desc(ng - 1, gbs[k], gss[k]).wait()
            out_desc(ng - _NBUF + k, obs[k], oss[k]).wait()

    return sc_gather_mean


def _build_tc_dense(b, d, nb, bx):
    eps = 1e-8

    def body(e0, e1, e2, r1, r2, r3, ssym, wqt, bq, wkt, bk, wvt, bv, wot, bo,
             g1, b1, g2, b2, g3, b3, c1t, c1b, c2t, c2b, cmt, cmb, out):
        def ln(t, gg, bb):
            m = jnp.mean(t, axis=1, keepdims=True)
            c = t - m
            v = jnp.mean(c * c, axis=1, keepdims=True)
            return c * lax.rsqrt(v + eps) * gg[:] + bb[:]

        dot = functools.partial(jnp.dot, preferred_element_type=jnp.float32)
        sm = ssym[:]
        e = [e0[:], e1[:], e2[:]]
        rr = [r1[:], r2[:], r3[:]]
        x = [dot(e[i], rr[i]) for i in range(3)]
        qn = [ln(xi, g1, b1) for xi in x]
        qh = [dot(qi, wqt[:]) + bq[:] for qi in qn]
        kh = [dot(xi, wkt[:]) + bk[:] for xi in x]
        vh = [dot(xi, wvt[:]) + bv[:] for xi in x]
        scale = 1.0 / math.sqrt(d // 4)
        sc = [[dot(qh[i] * kh[j], sm) * scale for j in range(3)] for i in range(3)]
        outs = []
        for i in range(3):
            mx = jnp.maximum(jnp.maximum(sc[i][0], sc[i][1]), sc[i][2])
            ex = [jnp.exp(sc[i][j] - mx) for j in range(3)]
            den = ex[0] + ex[1] + ex[2]
            o = (ex[0] * vh[0] + ex[1] * vh[1] + ex[2] * vh[2]) / den
            hi = qn[i] + dot(o, wot[:]) + bo[:]
            hi = ln(hi, g2, b2)
            fi = jnp.maximum(dot(hi, c1t[:]) + c1b[:], 0.0)
            fi = dot(fi, c2t[:]) + c2b[:]
            outs.append(ln(fi + hi, g3, b3))
        y = (dot(outs[0], cmt[0:d, :]) + dot(outs[1], cmt[d:2 * d, :])
             + dot(outs[2], cmt[2 * d:3 * d, :]) + cmb[:])
        out[:] = jnp.where(y > 0.0, y, jnp.exp(jnp.minimum(y, 0.0)) - 1.0)

    def espec(di):
        return pl.BlockSpec((bx, d), lambda i, di=di: (di * nb + i, 0))

    wspec = pl.BlockSpec((d, d), lambda i: (0, 0))
    vspec = pl.BlockSpec((1, d), lambda i: (0, 0))
    cspec = pl.BlockSpec((3 * d, d), lambda i: (0, 0))
    return pl.pallas_call(
        body,
        grid=(nb,),
        in_specs=[
            espec(0), espec(1), espec(2),
            wspec, wspec, wspec, wspec,
            wspec, vspec, wspec, vspec, wspec, vspec, wspec, vspec,
            vspec, vspec, vspec, vspec, vspec, vspec,
            wspec, vspec, wspec, vspec,
            cspec, vspec,
        ],
        out_specs=pl.BlockSpec((bx, d), lambda i: (i, 0)),
        out_shape=jax.ShapeDtypeStruct((b, d), jnp.float32),
    )


def kernel(features, nodes, to_neighs_dims, num_samples, r1, r2, r3,
           Wq, bq, Wk, bk, Wv, bv, Wo, bo,
           ln1_g, ln1_b, ln2_g, ln2_b, ln3_g, ln3_b,
           conv1_w, conv1_b, conv2_w, conv2_b, comb_w, comb_b):
    nd, b, s = to_neighs_dims.shape
    n_nodes, d = features.shape
    chunk_sizes = [6 * b // 10, 4 * b // 10]
    bx = 1000

    head = jnp.arange(d, dtype=jnp.int32) // (d // 4)
    ssym = (head[:, None] == head[None, :]).astype(jnp.float32)
    v = lambda t: t.reshape(1, d)
    inv = 1.0 / s
    r1s, r2s, r3s = r1 * inv, r2 * inv, r3 * inv
    outs = []
    c0 = 0
    for bc in chunk_sizes:
        assert bc % bx == 0
        nb = bc // bx
        rows = nd * bc
        rpw = -(-rows // _NW)
        rpw = -(-rpw // (_NBUF * _G)) * (_NBUF * _G)
        rows_pad = rpw * _NW
        tnc = lax.slice_in_dim(to_neighs_dims, c0, c0 + bc, axis=1)
        idx = tnc.astype(jnp.int32).reshape(rows * s)
        npad = (rows_pad - rows) * s
        idx = jnp.concatenate([idx, jnp.arange(npad, dtype=jnp.int32) % n_nodes])
        emb = _build_sc_gather_mean(d, s, rows_pad, rpw)(features, idx)
        outs.append(_build_tc_dense(bc, d, nb, bx)(
            emb, emb, emb,
            r1s, r2s, r3s, ssym,
            Wq.T, v(bq), Wk.T, v(bk), Wv.T, v(bv), Wo.T, v(bo),
            v(ln1_g), v(ln1_b), v(ln2_g), v(ln2_b), v(ln3_g), v(ln3_b),
            conv1_w.T, v(conv1_b), conv2_w.T, v(conv2_b),
            comb_w.T, v(comb_b),
        ))
        c0 += bc
    return jnp.concatenate(outs, axis=0)


# speedup vs baseline: 1.0306x; 1.0306x over previous
import functools
import math

import jax
import jax.numpy as jnp
from jax import lax
from jax.experimental import pallas as pl
from jax.experimental.pallas import tpu as pltpu
from jax.experimental.pallas import tpu_sc as plsc

_NC = 2
_NS = 16
_NW = _NC * _NS
_G = 8
_NBUF = 4


def _build_sc_gather_mean(d, s, rows_pad, rpw):
    gs = _G * s
    ng = rpw // _G
    nchunk = d // 16
    mesh = plsc.VectorSubcoreMesh(core_axis_name="c", subcore_axis_name="s")

    @functools.partial(
        pl.kernel,
        out_type=jax.ShapeDtypeStruct((rows_pad, d), jnp.float32),
        mesh=mesh,
        scratch_types=(
            [pltpu.VMEM((rpw * s,), jnp.int32)]
            + [pltpu.VMEM((gs, d), jnp.float32) for _ in range(_NBUF)]
            + [pltpu.VMEM((_G, d), jnp.float32) for _ in range(_NBUF)]
            + [pltpu.SemaphoreType.DMA for _ in range(2 * _NBUF)]
        ),
    )
    def sc_gather_mean(feat, idx, out, idx_v, *bufs):
        gbs = bufs[:_NBUF]
        obs = bufs[_NBUF:2 * _NBUF]
        gss = bufs[2 * _NBUF:3 * _NBUF]
        oss = bufs[3 * _NBUF:]
        wid = lax.axis_index("s") * _NC + lax.axis_index("c")
        row0 = wid * rpw
        pltpu.sync_copy(idx.at[pl.ds(row0 * s, rpw * s)], idx_v)

        def gather_desc(gi, gb, sem):
            off = pl.multiple_of(gi * gs, gs)
            return pltpu.make_async_copy(feat.at[idx_v.at[pl.ds(off, gs)]], gb, sem)

        def out_desc(gi, ob, sem):
            off = pl.multiple_of(row0 + gi * _G, _G)
            return pltpu.make_async_copy(ob, out.at[pl.ds(off, _G)], sem)

        def accumulate(gb, ob):
            def per_dest(di, carry):
                accs = [gb[di * s, pl.ds(ci * 16, 16)] for ci in range(nchunk)]
                for si in range(1, s):
                    for ci in range(nchunk):
                        accs[ci] = accs[ci] + gb[di * s + si, pl.ds(ci * 16, 16)]
                for ci in range(nchunk):
                    ob[di, pl.ds(ci * 16, 16)] = accs[ci]
                return carry

            lax.fori_loop(0, _G, per_dest, 0)

        def process(gi, k, first):
            gather_desc(gi, gbs[k], gss[k]).wait()
            if not first:
                out_desc(gi, obs[k], oss[k]).wait()
            accumulate(gbs[k], obs[k])
            nxt = gi + _NBUF if first else jnp.minimum(gi + _NBUF, ng - 1)
            gather_desc(nxt, gbs[k], gss[k]).start()
            out_desc(gi, obs[k], oss[k]).start()

        for k in range(_NBUF):
            gather_desc(k, gbs[k], gss[k]).start()
        for k in range(_NBUF):
            process(k, k, True)

        def body(t, carry):
            gi = t * _NBUF
            for k in range(_NBUF):
                process(gi + k, k, False)
            return carry

        lax.fori_loop(1, ng // _NBUF, body, 0)
        for k in range(_NBUF):
            gather_desc(ng - 1, gbs[k], gss[k]).wait()
            out_desc(ng - _NBUF + k, obs[k], oss[k]).wait()

    return sc_gather_mean


def _build_tc_dense(b, d, nb, bx):
    eps = 1e-8

    def body(e0, e1, e2, r1, r2, r3, ssym, wqt, bq, wkt, bk, wvt, bv, wot, bo,
             g1, b1, g2, b2, g3, b3, c1t, c1b, c2t, c2b, cmt, cmb, out):
        def ln(t, gg, bb):
            m = jnp.mean(t, axis=1, keepdims=True)
            c = t - m
            v = jnp.mean(c * c, axis=1, keepdims=True)
            return c * lax.rsqrt(v + eps) * gg[:] + bb[:]

        dot = functools.partial(jnp.dot, preferred_element_type=jnp.float32)
        sm = ssym[:]
        e = [e0[:], e1[:], e2[:]]
        rr = [r1[:], r2[:], r3[:]]
        x = [dot(e[i], rr[i]) for i in range(3)]
        qn = [ln(xi, g1, b1) for xi in x]
        qh = [dot(qi, wqt[:]) + bq[:] for qi in qn]
        kh = [dot(xi, wkt[:]) + bk[:] for xi in x]
        vh = [dot(xi, wvt[:]) + bv[:] for xi in x]
        scale = 1.0 / math.sqrt(d // 4)
        sc = [[dot(qh[i] * kh[j], sm) * scale for j in range(3)] for i in range(3)]
        outs = []
        for i in range(3):
            mx = jnp.maximum(jnp.maximum(sc[i][0], sc[i][1]), sc[i][2])
            ex = [jnp.exp(sc[i][j] - mx) for j in range(3)]
            den = ex[0] + ex[1] + ex[2]
            o = (ex[0] * vh[0] + ex[1] * vh[1] + ex[2] * vh[2]) / den
            hi = qn[i] + dot(o, wot[:]) + bo[:]
            hi = ln(hi, g2, b2)
            fi = jnp.maximum(dot(hi, c1t[:]) + c1b[:], 0.0)
            fi = dot(fi, c2t[:]) + c2b[:]
            outs.append(ln(fi + hi, g3, b3))
        y = (dot(outs[0], cmt[0:d, :]) + dot(outs[1], cmt[d:2 * d, :])
             + dot(outs[2], cmt[2 * d:3 * d, :]) + cmb[:])
        out[:] = jnp.where(y > 0.0, y, jnp.exp(jnp.minimum(y, 0.0)) - 1.0)

    def espec(di):
        return pl.BlockSpec((bx, d), lambda i, di=di: (di * nb + i, 0))

    wspec = pl.BlockSpec((d, d), lambda i: (0, 0))
    vspec = pl.BlockSpec((1, d), lambda i: (0, 0))
    cspec = pl.BlockSpec((3 * d, d), lambda i: (0, 0))
    return pl.pallas_call(
        body,
        grid=(nb,),
        in_specs=[
            espec(0), espec(1), espec(2),
            wspec, wspec, wspec, wspec,
            wspec, vspec, wspec, vspec, wspec, vspec, wspec, vspec,
            vspec, vspec, vspec, vspec, vspec, vspec,
            wspec, vspec, wspec, vspec,
            cspec, vspec,
        ],
        out_specs=pl.BlockSpec((bx, d), lambda i: (i, 0)),
        out_shape=jax.ShapeDtypeStruct((b, d), jnp.float32),
    )


def kernel(features, nodes, to_neighs_dims, num_samples, r1, r2, r3,
           Wq, bq, Wk, bk, Wv, bv, Wo, bo,
           ln1_g, ln1_b, ln2_g, ln2_b, ln3_g, ln3_b,
           conv1_w, conv1_b, conv2_w, conv2_b, comb_w, comb_b):
    nd, b, s = to_neighs_dims.shape
    n_nodes, d = features.shape
    chunk_sizes = [6 * b // 10, 4 * b // 10]
    bx = 1000

    head = jnp.arange(d, dtype=jnp.int32) // (d // 4)
    ssym = (head[:, None] == head[None, :]).astype(jnp.float32)
    v = lambda t: t.reshape(1, d)
    inv = 1.0 / s
    r1s, r2s, r3s = r1 * inv, r2 * inv, r3 * inv
    outs = []
    c0 = 0
    for bc in chunk_sizes:
        assert bc % bx == 0
        nb = bc // bx
        rows = nd * bc
        rpw = -(-rows // _NW)
        rpw = -(-rpw // (_NBUF * _G)) * (_NBUF * _G)
        rows_pad = rpw * _NW
        tnc = lax.slice_in_dim(to_neighs_dims, c0, c0 + bc, axis=1)
        idx = tnc.astype(jnp.int32).reshape(rows * s)
        npad = (rows_pad - rows) * s
        idx = jnp.concatenate([idx, jnp.arange(npad, dtype=jnp.int32) % n_nodes])
        emb = _build_sc_gather_mean(d, s, rows_pad, rpw)(features, idx)
        outs.append(_build_tc_dense(bc, d, nb, bx)(
            emb, emb, emb,
            r1s, r2s, r3s, ssym,
            Wq.T, v(bq), Wk.T, v(bk), Wv.T, v(bv), Wo.T, v(bo),
            v(ln1_g), v(ln1_b), v(ln2_g), v(ln2_b), v(ln3_g), v(ln3_b),
            conv1_w.T, v(conv1_b), conv2_w.T, v(conv2_b),
            comb_w.T, v(comb_b),
        ))
        c0 += bc
    return jnp.concatenate(outs, axis=0)


# speedup vs baseline: 1.0573x; 1.0259x over previous
import functools
import math

import jax
import jax.numpy as jnp
from jax import lax
from jax.experimental import pallas as pl
from jax.experimental.pallas import tpu as pltpu
from jax.experimental.pallas import tpu_sc as plsc

_NC = 2
_NS = 16
_NW = _NC * _NS
_G = 8
_NBUF = 4


def _build_sc_gather_mean(d, s, rows_pad, rpw):
    gs = _G * s
    ng = rpw // _G
    nchunk = d // 16
    mesh = plsc.VectorSubcoreMesh(core_axis_name="c", subcore_axis_name="s")

    @functools.partial(
        pl.kernel,
        out_type=jax.ShapeDtypeStruct((rows_pad, d), jnp.float32),
        mesh=mesh,
        scratch_types=(
            [pltpu.VMEM((rpw * s,), jnp.int32)]
            + [pltpu.VMEM((gs, d), jnp.float32) for _ in range(_NBUF)]
            + [pltpu.VMEM((_G, d), jnp.float32) for _ in range(_NBUF)]
            + [pltpu.SemaphoreType.DMA for _ in range(2 * _NBUF)]
        ),
    )
    def sc_gather_mean(feat, idx, out, idx_v, *bufs):
        gbs = bufs[:_NBUF]
        obs = bufs[_NBUF:2 * _NBUF]
        gss = bufs[2 * _NBUF:3 * _NBUF]
        oss = bufs[3 * _NBUF:]
        wid = lax.axis_index("s") * _NC + lax.axis_index("c")
        row0 = wid * rpw
        pltpu.sync_copy(idx.at[pl.ds(row0 * s, rpw * s)], idx_v)

        def gather_desc(gi, gb, sem):
            off = pl.multiple_of(gi * gs, gs)
            return pltpu.make_async_copy(feat.at[idx_v.at[pl.ds(off, gs)]], gb, sem)

        def out_desc(gi, ob, sem):
            off = pl.multiple_of(row0 + gi * _G, _G)
            return pltpu.make_async_copy(ob, out.at[pl.ds(off, _G)], sem)

        def accumulate(gb, ob):
            def per_dest(di, carry):
                accs = [gb[di * s, pl.ds(ci * 16, 16)] for ci in range(nchunk)]
                for si in range(1, s):
                    for ci in range(nchunk):
                        accs[ci] = accs[ci] + gb[di * s + si, pl.ds(ci * 16, 16)]
                for ci in range(nchunk):
                    ob[di, pl.ds(ci * 16, 16)] = accs[ci]
                return carry

            lax.fori_loop(0, _G, per_dest, 0)

        def process(gi, k, first):
            gather_desc(gi, gbs[k], gss[k]).wait()
            if not first:
                out_desc(gi, obs[k], oss[k]).wait()
            accumulate(gbs[k], obs[k])
            nxt = gi + _NBUF if first else jnp.minimum(gi + _NBUF, ng - 1)
            gather_desc(nxt, gbs[k], gss[k]).start()
            out_desc(gi, obs[k], oss[k]).start()

        for k in range(_NBUF):
            gather_desc(k, gbs[k], gss[k]).start()
        for k in range(_NBUF):
            process(k, k, True)

        def body(t, carry):
            gi = t * _NBUF
            for k in range(_NBUF):
                process(gi + k, k, False)
            return carry

        lax.fori_loop(1, ng // _NBUF, body, 0)
        for k in range(_NBUF):
            gather_desc(ng - 1, gbs[k], gss[k]).wait()
            out_desc(ng - _NBUF + k, obs[k], oss[k]).wait()

    return sc_gather_mean


def _build_tc_dense(b, d, nb, bx):
    eps = 1e-8

    def body(e0, e1, e2, r1, r2, r3, ssym, wqt, bq, wkt, bk, wvt, bv, wot, bo,
             g1, b1, g2, b2, g3, b3, c1t, c1b, c2t, c2b, cmt, cmb, out):
        def ln(t, gg, bb):
            m = jnp.mean(t, axis=1, keepdims=True)
            c = t - m
            v = jnp.mean(c * c, axis=1, keepdims=True)
            return c * lax.rsqrt(v + eps) * gg[:] + bb[:]

        dot = functools.partial(jnp.dot, preferred_element_type=jnp.float32)
        sm = ssym[:]
        e = [e0[:], e1[:], e2[:]]
        rr = [r1[:], r2[:], r3[:]]
        x = [dot(e[i], rr[i]) for i in range(3)]
        qn = [ln(xi, g1, b1) for xi in x]
        qh = [dot(qi, wqt[:]) + bq[:] for qi in qn]
        kh = [dot(xi, wkt[:]) + bk[:] for xi in x]
        vh = [dot(xi, wvt[:]) + bv[:] for xi in x]
        scale = 1.0 / math.sqrt(d // 4)
        sc = [[dot(qh[i] * kh[j], sm) * scale for j in range(3)] for i in range(3)]
        outs = []
        for i in range(3):
            mx = jnp.maximum(jnp.maximum(sc[i][0], sc[i][1]), sc[i][2])
            ex = [jnp.exp(sc[i][j] - mx) for j in range(3)]
            den = ex[0] + ex[1] + ex[2]
            o = (ex[0] * vh[0] + ex[1] * vh[1] + ex[2] * vh[2]) / den
            hi = qn[i] + dot(o, wot[:]) + bo[:]
            hi = ln(hi, g2, b2)
            fi = jnp.maximum(dot(hi, c1t[:]) + c1b[:], 0.0)
            fi = dot(fi, c2t[:]) + c2b[:]
            outs.append(ln(fi + hi, g3, b3))
        y = (dot(outs[0], cmt[0:d, :]) + dot(outs[1], cmt[d:2 * d, :])
             + dot(outs[2], cmt[2 * d:3 * d, :]) + cmb[:])
        out[:] = jnp.where(y > 0.0, y, jnp.exp(jnp.minimum(y, 0.0)) - 1.0)

    def espec(di):
        return pl.BlockSpec((bx, d), lambda i, di=di: (di * nb + i, 0))

    wspec = pl.BlockSpec((d, d), lambda i: (0, 0))
    vspec = pl.BlockSpec((1, d), lambda i: (0, 0))
    cspec = pl.BlockSpec((3 * d, d), lambda i: (0, 0))
    return pl.pallas_call(
        body,
        grid=(nb,),
        in_specs=[
            espec(0), espec(1), espec(2),
            wspec, wspec, wspec, wspec,
            wspec, vspec, wspec, vspec, wspec, vspec, wspec, vspec,
            vspec, vspec, vspec, vspec, vspec, vspec,
            wspec, vspec, wspec, vspec,
            cspec, vspec,
        ],
        out_specs=pl.BlockSpec((bx, d), lambda i: (i, 0)),
        out_shape=jax.ShapeDtypeStruct((b, d), jnp.float32),
    )


def kernel(features, nodes, to_neighs_dims, num_samples, r1, r2, r3,
           Wq, bq, Wk, bk, Wv, bv, Wo, bo,
           ln1_g, ln1_b, ln2_g, ln2_b, ln3_g, ln3_b,
           conv1_w, conv1_b, conv2_w, conv2_b, comb_w, comb_b):
    nd, b, s = to_neighs_dims.shape
    n_nodes, d = features.shape
    chunk_sizes = [6 * b // 10, 4 * b // 10]
    bx = 2000

    head = jnp.arange(d, dtype=jnp.int32) // (d // 4)
    ssym = (head[:, None] == head[None, :]).astype(jnp.float32)
    v = lambda t: t.reshape(1, d)
    inv = 1.0 / s
    r1s, r2s, r3s = r1 * inv, r2 * inv, r3 * inv
    outs = []
    c0 = 0
    for bc in chunk_sizes:
        assert bc % bx == 0
        nb = bc // bx
        rows = nd * bc
        rpw = -(-rows // _NW)
        rpw = -(-rpw // (_NBUF * _G)) * (_NBUF * _G)
        rows_pad = rpw * _NW
        tnc = lax.slice_in_dim(to_neighs_dims, c0, c0 + bc, axis=1)
        idx = tnc.astype(jnp.int32).reshape(rows * s)
        npad = (rows_pad - rows) * s
        idx = jnp.concatenate([idx, jnp.arange(npad, dtype=jnp.int32) % n_nodes])
        emb = _build_sc_gather_mean(d, s, rows_pad, rpw)(features, idx)
        outs.append(_build_tc_dense(bc, d, nb, bx)(
            emb, emb, emb,
            r1s, r2s, r3s, ssym,
            Wq.T, v(bq), Wk.T, v(bk), Wv.T, v(bv), Wo.T, v(bo),
            v(ln1_g), v(ln1_b), v(ln2_g), v(ln2_b), v(ln3_g), v(ln3_b),
            conv1_w.T, v(conv1_b), conv2_w.T, v(conv2_b),
            comb_w.T, v(comb_b),
        ))
        c0 += bc
    return jnp.concatenate(outs, axis=0)
